# Initial kernel scaffold; baseline (speedup 1.0000x reference)
import jax
import jax.numpy as jnp
from jax.experimental import pallas as pl


def kernel(x, weight):
    raise NotImplementedError("write your pallas kernel here")


# speedup vs baseline: 1.4614x; 1.4614x over previous
import functools

import jax
import jax.numpy as jnp
from jax import lax
from jax.experimental import pallas as pl
from jax.experimental.pallas import tpu as pltpu
from jax.experimental.pallas import tpu_sc as plsc

_NUM_CORES = 2
_NUM_SUBCORES = 16
_NUM_WORKERS = _NUM_CORES * _NUM_SUBCORES

_CHUNK = 1024


def _build(total_rows: int, dim: int):
  rows_per_worker = total_rows // _NUM_WORKERS
  n_chunks = rows_per_worker // _CHUNK
  assert rows_per_worker % _CHUNK == 0

  mesh = plsc.VectorSubcoreMesh(core_axis_name="c", subcore_axis_name="s")

  @functools.partial(
      pl.kernel,
      mesh=mesh,
      out_type=jax.ShapeDtypeStruct((total_rows, dim), jnp.float32),
      scratch_types=[
          pltpu.VMEM((_CHUNK,), jnp.int32),
          pltpu.VMEM((_CHUNK, dim), jnp.float32),
          pltpu.SemaphoreType.DMA,
      ],
      compiler_params=pltpu.CompilerParams(use_tc_tiling_on_sc=False),
  )
  def emb(idx_hbm, w_hbm, out_hbm, idx_v, rows_v, sem):
    wid = lax.axis_index("s") * _NUM_CORES + lax.axis_index("c")
    base = wid * rows_per_worker

    def step(j, carry):
      off = base + j * _CHUNK
      pltpu.sync_copy(idx_hbm.at[pl.ds(off, _CHUNK)], idx_v)
      pltpu.async_copy(w_hbm.at[idx_v], rows_v, sem).wait()
      pltpu.sync_copy(rows_v, out_hbm.at[pl.ds(off, _CHUNK)])
      return carry

    lax.fori_loop(0, n_chunks, step, 0)

  return emb


@jax.jit
def kernel(x, weight):
  b, t = x.shape
  dim = weight.shape[1]
  flat = x.reshape(-1)
  emb = _build(flat.shape[0], dim)
  out = emb(flat, weight)
  return out.reshape(b, t, dim)


# speedup vs baseline: 1.5035x; 1.0288x over previous
import functools

import jax
import jax.numpy as jnp
from jax import lax
from jax.experimental import pallas as pl
from jax.experimental.pallas import tpu as pltpu
from jax.experimental.pallas import tpu_sc as plsc

_NUM_CORES = 2
_NUM_SUBCORES = 16
_NUM_WORKERS = _NUM_CORES * _NUM_SUBCORES

_CHUNK = 256
_NBUF = 5


def _build(total_rows: int, dim: int):
  rows_per_worker = total_rows // _NUM_WORKERS
  n_chunks = rows_per_worker // _CHUNK
  n_outer = n_chunks // _NBUF
  assert rows_per_worker % _CHUNK == 0 and n_chunks % _NBUF == 0
  assert n_outer % 2 == 0

  mesh = plsc.VectorSubcoreMesh(core_axis_name="c", subcore_axis_name="s")

  @functools.partial(
      pl.kernel,
      mesh=mesh,
      out_type=jax.ShapeDtypeStruct((total_rows, dim), jnp.float32),
      scratch_types=[
          pltpu.VMEM((n_chunks, _CHUNK), jnp.int32),
          pltpu.VMEM((_NBUF, 2, _CHUNK, dim), jnp.float32),
          pltpu.SemaphoreType.DMA((_NBUF,)),
          pltpu.SemaphoreType.DMA((_NBUF, 2)),
      ],
      compiler_params=pltpu.CompilerParams(use_tc_tiling_on_sc=False),
  )
  def emb(idx_hbm, w_hbm, out_hbm, idx_v, rows_v, gsem, ssem):
    wid = lax.axis_index("s") * _NUM_CORES + lax.axis_index("c")
    base = wid * n_chunks

    pltpu.sync_copy(idx_hbm.at[pl.ds(base, n_chunks)], idx_v)

    def gather_cp(j, b, p):
      return pltpu.make_async_copy(
          w_hbm.at[idx_v.at[j]], rows_v.at[b, p], gsem.at[b])

    def store_cp(j, b, p):
      return pltpu.make_async_copy(
          rows_v.at[b, p],
          out_hbm.at[pl.ds((base + j) * _CHUNK, _CHUNK)],
          ssem.at[b, p])

    for b in range(_NBUF):
      gather_cp(b, b, 0).start()

    def round_body(r, p):
      for b in range(_NBUF):
        j = r * _NBUF + b
        gather_cp(j, b, p).wait()
        store_cp(j, b, p).start()

        @pl.when(jnp.logical_and(r >= 1, r < n_outer - 1))
        def _():
          store_cp(j - _NBUF, b, 1 - p).wait()

        @pl.when(r < n_outer - 1)
        def _():
          gather_cp(j + _NBUF, b, 1 - p).start()

    def two_rounds(rr, carry):
      round_body(rr * 2, 0)
      round_body(rr * 2 + 1, 1)
      return carry

    lax.fori_loop(0, n_outer // 2, two_rounds, 0)

    for b in range(_NBUF):
      store_cp((n_outer - 2) * _NBUF + b, b, 0).wait()
      store_cp((n_outer - 1) * _NBUF + b, b, 1).wait()

  return emb


@jax.jit
def kernel(x, weight):
  b, t = x.shape
  dim = weight.shape[1]
  flat = x.reshape(-1, _CHUNK)
  emb = _build(x.size, dim)
  out = emb(flat, weight)
  return out.reshape(b, t, dim)
